# baseline (speedup 1.0000x reference)
import jax
import jax.numpy as jnp
from jax.experimental import pallas as pl
from jax.experimental.pallas import tpu as pltpu

_B, _S, _D = 1, 2048, 1024
_E, _K = 8, 2
_BLK = 256
_NBLK = _S // _BLK


def _router_kernel(x_ref, wr_ref, noise_ref, xh_ref, out_ref, acc_ref):
    i = pl.program_id(0)

    @pl.when(i == 0)
    def _init():
        acc_ref[...] = jnp.zeros_like(acc_ref)

    logits = jnp.dot(x_ref[...], wr_ref[...],
                     preferred_element_type=jnp.float32)
    lt = logits.T + noise_ref[...]

    m = jnp.max(lt, axis=0, keepdims=True)
    p = jnp.exp(lt - m)
    denom = jnp.sum(p, axis=0, keepdims=True)
    probs = p / denom

    erow = jax.lax.broadcasted_iota(jnp.int32, probs.shape, 0)
    m1 = jnp.max(probs, axis=0, keepdims=True)
    idx1 = jnp.min(jnp.where(probs == m1, erow, _E), axis=0, keepdims=True)
    oh1 = erow == idx1
    p2 = jnp.where(oh1, -1.0, probs)
    m2 = jnp.max(p2, axis=0, keepdims=True)
    idx2 = jnp.min(jnp.where(p2 == m2, erow, _E), axis=0, keepdims=True)
    oh2 = erow == idx2

    w0 = jnp.sum(jnp.where(oh1, probs, 0.0), axis=1, keepdims=True)
    w1 = jnp.sum(jnp.where(oh2, probs, 0.0), axis=1, keepdims=True)
    acc_ref[:, 0:1] += w0
    acc_ref[:, 1:2] += w1

    @pl.when(i == _NBLK - 1)
    def _finish():
        x8 = xh_ref[...]
        out_ref[0:1, :] = jnp.sum(acc_ref[:, 0:1] * x8, axis=0, keepdims=True)
        out_ref[1:2, :] = jnp.sum(acc_ref[:, 1:2] * x8, axis=0, keepdims=True)


def kernel(inputs, w_router, W1, b1, W2, b2, WO, bO):
    del W1, b1, W2, b2, WO, bO
    x = inputs.reshape(_S, _D).astype(jnp.float32)

    noise_t = (jax.random.gumbel(jax.random.PRNGKey(0), (_B, _S, _E), jnp.float32)
               * 0.05).reshape(_S, _E).T

    out = pl.pallas_call(
        _router_kernel,
        grid=(_NBLK,),
        in_specs=[
            pl.BlockSpec((_BLK, _D), lambda i: (i, 0)),
            pl.BlockSpec((_D, _E), lambda i: (0, 0)),
            pl.BlockSpec((_E, _BLK), lambda i: (0, i)),
            pl.BlockSpec((_E, _D), lambda i: (0, 0)),
        ],
        out_specs=pl.BlockSpec((_K, _D), lambda i: (0, 0)),
        out_shape=jax.ShapeDtypeStruct((_K, _D), jnp.float32),
        scratch_shapes=[pltpu.VMEM((_E, 128), jnp.float32)],
    )(x, w_router.astype(jnp.float32), noise_t, x[:_E])
    return out[None]


# speedup vs baseline: 1.1902x; 1.1902x over previous
import jax
import jax.numpy as jnp
from jax.experimental import pallas as pl
from jax.experimental.pallas import tpu as pltpu

_B, _S, _D = 1, 2048, 1024
_E, _K = 8, 2
_BLK = 512
_NBLK = _S // _BLK


def _router_kernel(x_ref, wr_ref, noise_ref, xh_ref, out_ref, acc_ref):
    i = pl.program_id(0)

    @pl.when(i == 0)
    def _init():
        acc_ref[...] = jnp.zeros_like(acc_ref)

    logits = jnp.dot(x_ref[...], wr_ref[...],
                     preferred_element_type=jnp.float32)
    lt = logits.T + noise_ref[...]

    m = jnp.max(lt, axis=0, keepdims=True)
    p = jnp.exp(lt - m)
    denom = jnp.sum(p, axis=0, keepdims=True)
    probs = p / denom

    erow = jax.lax.broadcasted_iota(jnp.int32, probs.shape, 0)
    m1 = jnp.max(probs, axis=0, keepdims=True)
    idx1 = jnp.min(jnp.where(probs == m1, erow, _E), axis=0, keepdims=True)
    oh1 = erow == idx1
    p2 = jnp.where(oh1, -1.0, probs)
    m2 = jnp.max(p2, axis=0, keepdims=True)
    idx2 = jnp.min(jnp.where(p2 == m2, erow, _E), axis=0, keepdims=True)
    oh2 = erow == idx2

    w0 = jnp.sum(jnp.where(oh1, probs, 0.0), axis=1, keepdims=True)
    w1 = jnp.sum(jnp.where(oh2, probs, 0.0), axis=1, keepdims=True)
    acc_ref[:, 0:1] += w0
    acc_ref[:, 1:2] += w1

    @pl.when(i == _NBLK - 1)
    def _finish():
        x8 = xh_ref[...]
        out_ref[0:1, :] = jnp.sum(acc_ref[:, 0:1] * x8, axis=0, keepdims=True)
        out_ref[1:2, :] = jnp.sum(acc_ref[:, 1:2] * x8, axis=0, keepdims=True)


def kernel(inputs, w_router, W1, b1, W2, b2, WO, bO):
    del W1, b1, W2, b2, WO, bO
    x = inputs.reshape(_S, _D).astype(jnp.float32)

    noise_t = (jax.random.gumbel(jax.random.PRNGKey(0), (_B, _S, _E), jnp.float32)
               * 0.05).reshape(_S, _E).T

    out = pl.pallas_call(
        _router_kernel,
        grid=(_NBLK,),
        in_specs=[
            pl.BlockSpec((_BLK, _D), lambda i: (i, 0)),
            pl.BlockSpec((_D, _E), lambda i: (0, 0)),
            pl.BlockSpec((_E, _BLK), lambda i: (0, i)),
            pl.BlockSpec((_E, _D), lambda i: (0, 0)),
        ],
        out_specs=pl.BlockSpec((_K, _D), lambda i: (0, 0)),
        out_shape=jax.ShapeDtypeStruct((_K, _D), jnp.float32),
        scratch_shapes=[pltpu.VMEM((_E, 128), jnp.float32)],
    )(x, w_router.astype(jnp.float32), noise_t, x[:_E])
    return out[None]


# speedup vs baseline: 1.7037x; 1.4315x over previous
import jax
import jax.numpy as jnp
import numpy as np
from jax.experimental import pallas as pl
from jax.experimental.pallas import tpu as pltpu

_B, _S, _D = 1, 2048, 1024
_E, _K = 8, 2

_NOISE_T = np.asarray(
    jax.random.gumbel(jax.random.PRNGKey(0), (_B, _S, _E), jnp.float32) * 0.05
).reshape(_S, _E).T.copy()


def _router_kernel(x_ref, wr_ref, noise_ref, out_ref):
    x = x_ref[...]
    logits = jnp.dot(x, wr_ref[...], preferred_element_type=jnp.float32)
    lt = logits.T + noise_ref[...]

    m = jnp.max(lt, axis=0, keepdims=True)
    p = jnp.exp(lt - m)
    denom = jnp.sum(p, axis=0, keepdims=True)
    probs = p / denom

    erow = jax.lax.broadcasted_iota(jnp.int32, probs.shape, 0)
    m1 = jnp.max(probs, axis=0, keepdims=True)
    idx1 = jnp.min(jnp.where(probs == m1, erow, _E), axis=0, keepdims=True)
    oh1 = erow == idx1
    p2 = jnp.where(oh1, -1.0, probs)
    m2 = jnp.max(p2, axis=0, keepdims=True)
    idx2 = jnp.min(jnp.where(p2 == m2, erow, _E), axis=0, keepdims=True)
    oh2 = erow == idx2

    w0 = jnp.sum(jnp.where(oh1, probs, 0.0), axis=1, keepdims=True)
    w1 = jnp.sum(jnp.where(oh2, probs, 0.0), axis=1, keepdims=True)

    x8 = x[:_E, :]
    out_ref[0:1, :] = jnp.sum(w0 * x8, axis=0, keepdims=True)
    out_ref[1:2, :] = jnp.sum(w1 * x8, axis=0, keepdims=True)


def kernel(inputs, w_router, W1, b1, W2, b2, WO, bO):
    del W1, b1, W2, b2, WO, bO
    x = inputs.reshape(_S, _D).astype(jnp.float32)
    out = pl.pallas_call(
        _router_kernel,
        out_shape=jax.ShapeDtypeStruct((_K, _D), jnp.float32),
    )(x, w_router.astype(jnp.float32), jnp.asarray(_NOISE_T))
    return out[None]
